# baseline (speedup 1.0000x reference)
import jax
import jax.numpy as jnp
from jax import lax
from jax.experimental import pallas as pl

_NE = 1024
_D = 64
_B = 16
_P = 1024
_NPIX = _B * _P
_NELEM = _B * _D * _P


def _vq_body(z_ref, wt_ref, w2x_ref, w2_ref, iota_ref, out_ref, idx_ref,
             cnt_ref, loss_ref, perp_ref, util_ref):
    b = pl.program_id(0)
    zr = z_ref[0]
    wt = wt_ref[...]

    z2 = jnp.sum(zr * zr, axis=0, keepdims=True)
    m2 = lax.dot(w2x_ref[...], zr,
                 preferred_element_type=jnp.float32)
    d = (z2 - m2) + w2_ref[...]

    dmin = jnp.min(d, axis=0, keepdims=True)
    s = jnp.where(d == dmin, iota_ref[...], 1e9)
    sf = jnp.min(s, axis=0, keepdims=True)
    idx_ref[0] = sf.astype(jnp.int32)

    e = jnp.where(s == sf, 1.0, 0.0)
    q = lax.dot(wt, e, preferred_element_type=jnp.float32)
    out_ref[0] = zr + (q - zr)

    ls = jnp.sum((q - zr) ** 2, axis=(0, 1), keepdims=True)
    cnt = jnp.sum(e, axis=1, keepdims=True)

    @pl.when(b == 0)
    def _():
        cnt_ref[...] = cnt
        loss_ref[...] = ls

    @pl.when(b > 0)
    def _():
        cnt_ref[...] += cnt
        loss_ref[...] += ls

    @pl.when(b == _B - 1)
    def _():
        mean = loss_ref[...] * (1.0 / _NELEM)
        loss_ref[...] = mean + 0.25 * mean
        p = cnt_ref[...] * (1.0 / _NPIX)
        plog = p * jnp.log(p + 1e-10)
        perp_ref[...] = jnp.exp(-jnp.sum(plog, axis=(0, 1), keepdims=True))
        util_ref[...] = jnp.sum(jnp.where(p > 0, 1.0, 0.0),
                                axis=(0, 1), keepdims=True) * (1.0 / _NE)


def _vq_pallas(z3, wt, w2x, w2, iotaf):
    return pl.pallas_call(
        _vq_body,
        grid=(_B,),
        in_specs=[
            pl.BlockSpec((1, _D, _P), lambda b: (b, 0, 0)),
            pl.BlockSpec((_D, _NE), lambda b: (0, 0)),
            pl.BlockSpec((_NE, _D), lambda b: (0, 0)),
            pl.BlockSpec((_NE, 1), lambda b: (0, 0)),
            pl.BlockSpec((_NE, _P), lambda b: (0, 0)),
        ],
        out_specs=[
            pl.BlockSpec((1, _D, _P), lambda b: (b, 0, 0)),
            pl.BlockSpec((1, 1, _P), lambda b: (b, 0, 0)),
            pl.BlockSpec((_NE, 1), lambda b: (0, 0)),
            pl.BlockSpec((1, 1), lambda b: (0, 0)),
            pl.BlockSpec((1, 1), lambda b: (0, 0)),
            pl.BlockSpec((1, 1), lambda b: (0, 0)),
        ],
        out_shape=[
            jax.ShapeDtypeStruct((_B, _D, _P), jnp.float32),
            jax.ShapeDtypeStruct((_B, 1, _P), jnp.int32),
            jax.ShapeDtypeStruct((_NE, 1), jnp.float32),
            jax.ShapeDtypeStruct((1, 1), jnp.float32),
            jax.ShapeDtypeStruct((1, 1), jnp.float32),
            jax.ShapeDtypeStruct((1, 1), jnp.float32),
        ],
    )(z3, wt, w2x, w2, iotaf)


def kernel(z, W):
    z3 = z.reshape(_B, _D, _P)
    w2 = jnp.sum(W ** 2, axis=1, keepdims=True)
    iotaf = lax.broadcasted_iota(jnp.float32, (_NE, _P), 0)
    qst3, idx3, _cnt, loss, perp, util = _vq_pallas(z3, W.T, W + W, w2, iotaf)
    quantized_st = qst3.reshape(z.shape)
    encoding_indices = idx3.reshape(_NPIX)
    return (quantized_st, loss[0, 0], perp[0, 0], util[0, 0],
            encoding_indices)


# speedup vs baseline: 1.0569x; 1.0569x over previous
import jax
import jax.numpy as jnp
from jax import lax
from jax.experimental import pallas as pl

_NE = 1024
_D = 64
_B = 16
_P = 1024
_NPIX = _B * _P
_NELEM = _B * _D * _P


def _vq_body(z_ref, wt_ref, w2x_ref, w2_ref, out_ref, idx_ref,
             cnt_ref, loss_ref, perp_ref, util_ref):
    b = pl.program_id(0)
    zr = z_ref[0]
    wt = wt_ref[...]

    z2 = jnp.sum(zr * zr, axis=0, keepdims=True)
    m2 = lax.dot(w2x_ref[...], zr,
                 preferred_element_type=jnp.float32)
    d = (z2 - m2) + w2_ref[...]

    dmin = jnp.min(d, axis=0, keepdims=True)
    iotaf = lax.broadcasted_iota(jnp.int32, (_NE, _P), 0).astype(jnp.float32)
    s = jnp.where(d == dmin, iotaf, 1e9)
    sf = jnp.min(s, axis=0, keepdims=True)
    idx_ref[0] = sf.astype(jnp.int32)

    e = jnp.where(s == sf, 1.0, 0.0)
    q = lax.dot(wt, e, preferred_element_type=jnp.float32)
    out_ref[0] = zr + (q - zr)

    ls = jnp.sum((q - zr) ** 2, axis=(0, 1), keepdims=True)
    cnt = jnp.sum(e, axis=1, keepdims=True)

    @pl.when(b == 0)
    def _():
        cnt_ref[...] = cnt
        loss_ref[...] = ls

    @pl.when(b > 0)
    def _():
        cnt_ref[...] += cnt
        loss_ref[...] += ls

    @pl.when(b == _B - 1)
    def _():
        mean = loss_ref[...] * (1.0 / _NELEM)
        loss_ref[...] = mean + 0.25 * mean
        p = cnt_ref[...] * (1.0 / _NPIX)
        plog = p * jnp.log(p + 1e-10)
        perp_ref[...] = jnp.exp(-jnp.sum(plog, axis=(0, 1), keepdims=True))
        util_ref[...] = jnp.sum(jnp.where(p > 0, 1.0, 0.0),
                                axis=(0, 1), keepdims=True) * (1.0 / _NE)


def _vq_pallas(z3, wt, w2x, w2):
    return pl.pallas_call(
        _vq_body,
        grid=(_B,),
        in_specs=[
            pl.BlockSpec((1, _D, _P), lambda b: (b, 0, 0)),
            pl.BlockSpec((_D, _NE), lambda b: (0, 0)),
            pl.BlockSpec((_NE, _D), lambda b: (0, 0)),
            pl.BlockSpec((_NE, 1), lambda b: (0, 0)),
        ],
        out_specs=[
            pl.BlockSpec((1, _D, _P), lambda b: (b, 0, 0)),
            pl.BlockSpec((1, 1, _P), lambda b: (b, 0, 0)),
            pl.BlockSpec((_NE, 1), lambda b: (0, 0)),
            pl.BlockSpec((1, 1), lambda b: (0, 0)),
            pl.BlockSpec((1, 1), lambda b: (0, 0)),
            pl.BlockSpec((1, 1), lambda b: (0, 0)),
        ],
        out_shape=[
            jax.ShapeDtypeStruct((_B, _D, _P), jnp.float32),
            jax.ShapeDtypeStruct((_B, 1, _P), jnp.int32),
            jax.ShapeDtypeStruct((_NE, 1), jnp.float32),
            jax.ShapeDtypeStruct((1, 1), jnp.float32),
            jax.ShapeDtypeStruct((1, 1), jnp.float32),
            jax.ShapeDtypeStruct((1, 1), jnp.float32),
        ],
    )(z3, wt, w2x, w2)


def kernel(z, W):
    z3 = z.reshape(_B, _D, _P)
    w2 = jnp.sum(W ** 2, axis=1, keepdims=True)
    qst3, idx3, _cnt, loss, perp, util = _vq_pallas(z3, W.T, W + W, w2)
    quantized_st = qst3.reshape(z.shape)
    encoding_indices = idx3.reshape(_NPIX)
    return (quantized_st, loss[0, 0], perp[0, 0], util[0, 0],
            encoding_indices)


# speedup vs baseline: 1.0873x; 1.0288x over previous
import jax
import jax.numpy as jnp
from jax import lax
from jax.experimental import pallas as pl

_NE = 1024
_D = 64
_B = 16
_P = 1024
_BLK = 4
_NSTEP = _B // _BLK
_NPIX = _B * _P
_NELEM = _B * _D * _P


def _vq_body(z_ref, wt_ref, w2x_ref, w2_ref, out_ref, idx_ref,
             cnt_ref, loss_ref, perp_ref, util_ref):
    b = pl.program_id(0)
    wt = wt_ref[...]
    w2x = w2x_ref[...]
    w2 = w2_ref[...]

    zrs, m2s, z2s = [], [], []
    for j in range(_BLK):
        zr = z_ref[j]
        zrs.append(zr)
        z2s.append(jnp.sum(zr * zr, axis=0, keepdims=True))
        m2s.append(lax.dot(w2x, zr,
                           preferred_element_type=jnp.float32))

    iotaf = lax.broadcasted_iota(jnp.int32, (_NE, _P), 0).astype(jnp.float32)
    ls_acc = None
    cnt_acc = None
    for j in range(_BLK):
        zr = zrs[j]
        d = (z2s[j] - m2s[j]) + w2
        dmin = jnp.min(d, axis=0, keepdims=True)
        s = jnp.where(d == dmin, iotaf, 1e9)
        sf = jnp.min(s, axis=0, keepdims=True)
        idx_ref[j] = sf.astype(jnp.int32)

        e = jnp.where(s == sf, 1.0, 0.0)
        q = lax.dot(wt, e, preferred_element_type=jnp.float32)
        out_ref[j] = zr + (q - zr)

        ls = jnp.sum((q - zr) ** 2, axis=(0, 1), keepdims=True)
        cnt = jnp.sum(e, axis=1, keepdims=True)
        ls_acc = ls if ls_acc is None else ls_acc + ls
        cnt_acc = cnt if cnt_acc is None else cnt_acc + cnt

    @pl.when(b == 0)
    def _():
        cnt_ref[...] = cnt_acc
        loss_ref[...] = ls_acc

    @pl.when(b > 0)
    def _():
        cnt_ref[...] += cnt_acc
        loss_ref[...] += ls_acc

    @pl.when(b == _NSTEP - 1)
    def _():
        mean = loss_ref[...] * (1.0 / _NELEM)
        loss_ref[...] = mean + 0.25 * mean
        p = cnt_ref[...] * (1.0 / _NPIX)
        plog = p * jnp.log(p + 1e-10)
        perp_ref[...] = jnp.exp(-jnp.sum(plog, axis=(0, 1), keepdims=True))
        util_ref[...] = jnp.sum(jnp.where(p > 0, 1.0, 0.0),
                                axis=(0, 1), keepdims=True) * (1.0 / _NE)


def _vq_pallas(z3, wt, w2x, w2):
    return pl.pallas_call(
        _vq_body,
        grid=(_NSTEP,),
        in_specs=[
            pl.BlockSpec((_BLK, _D, _P), lambda b: (b, 0, 0)),
            pl.BlockSpec((_D, _NE), lambda b: (0, 0)),
            pl.BlockSpec((_NE, _D), lambda b: (0, 0)),
            pl.BlockSpec((_NE, 1), lambda b: (0, 0)),
        ],
        out_specs=[
            pl.BlockSpec((_BLK, _D, _P), lambda b: (b, 0, 0)),
            pl.BlockSpec((_BLK, 1, _P), lambda b: (b, 0, 0)),
            pl.BlockSpec((_NE, 1), lambda b: (0, 0)),
            pl.BlockSpec((1, 1), lambda b: (0, 0)),
            pl.BlockSpec((1, 1), lambda b: (0, 0)),
            pl.BlockSpec((1, 1), lambda b: (0, 0)),
        ],
        out_shape=[
            jax.ShapeDtypeStruct((_B, _D, _P), jnp.float32),
            jax.ShapeDtypeStruct((_B, 1, _P), jnp.int32),
            jax.ShapeDtypeStruct((_NE, 1), jnp.float32),
            jax.ShapeDtypeStruct((1, 1), jnp.float32),
            jax.ShapeDtypeStruct((1, 1), jnp.float32),
            jax.ShapeDtypeStruct((1, 1), jnp.float32),
        ],
    )(z3, wt, w2x, w2)


def kernel(z, W):
    z3 = z.reshape(_B, _D, _P)
    w2 = jnp.sum(W ** 2, axis=1, keepdims=True)
    qst3, idx3, _cnt, loss, perp, util = _vq_pallas(z3, W.T, W + W, w2)
    quantized_st = qst3.reshape(z.shape)
    encoding_indices = idx3.reshape(_NPIX)
    return (quantized_st, loss[0, 0], perp[0, 0], util[0, 0],
            encoding_indices)


# speedup vs baseline: 1.1000x; 1.0116x over previous
import jax
import jax.numpy as jnp
from jax import lax
from jax.experimental import pallas as pl

_NE = 1024
_D = 64
_B = 16
_P = 1024
_BLK = 4
_NSTEP = _B // _BLK
_NPIX = _B * _P
_NELEM = _B * _D * _P


def _vq_body(z_ref, wt_ref, w2x_ref, w2_ref, out_ref, idx_ref,
             cnt_ref, loss_ref, perp_ref, util_ref):
    b = pl.program_id(0)
    wt = wt_ref[...]
    w2x = w2x_ref[...]
    w2 = w2_ref[...]

    zrs, m2s, z2s = [], [], []
    for j in range(_BLK):
        zr = z_ref[j]
        zrs.append(zr)
        z2s.append(jnp.sum(zr * zr, axis=0, keepdims=True))
        m2s.append(lax.dot(w2x, zr,
                           preferred_element_type=jnp.float32))

    iotaf = lax.broadcasted_iota(jnp.int32, (_NE, _P), 0).astype(jnp.float32)
    ls_acc = None
    cnt_acc = None
    for j in range(_BLK):
        zr = zrs[j]
        d = (z2s[j] - m2s[j]) + w2
        dmin = jnp.min(d, axis=0, keepdims=True)
        s = jnp.where(d == dmin, iotaf, 1e9)
        sf = jnp.min(s, axis=0, keepdims=True)
        idx_ref[j] = sf.astype(jnp.int32)

        e = jnp.where(s == sf, 1.0, 0.0)
        q = lax.dot(wt, e, preferred_element_type=jnp.float32)
        out_ref[j] = zr + (q - zr)

        ls = jnp.sum(dmin, axis=(0, 1), keepdims=True)
        cnt = jnp.sum(e, axis=1, keepdims=True)
        ls_acc = ls if ls_acc is None else ls_acc + ls
        cnt_acc = cnt if cnt_acc is None else cnt_acc + cnt

    @pl.when(b == 0)
    def _():
        cnt_ref[...] = cnt_acc
        loss_ref[...] = ls_acc

    @pl.when(b > 0)
    def _():
        cnt_ref[...] += cnt_acc
        loss_ref[...] += ls_acc

    @pl.when(b == _NSTEP - 1)
    def _():
        mean = loss_ref[...] * (1.0 / _NELEM)
        loss_ref[...] = mean + 0.25 * mean
        p = cnt_ref[...] * (1.0 / _NPIX)
        plog = p * jnp.log(p + 1e-10)
        perp_ref[...] = jnp.exp(-jnp.sum(plog, axis=(0, 1), keepdims=True))
        util_ref[...] = jnp.sum(jnp.where(p > 0, 1.0, 0.0),
                                axis=(0, 1), keepdims=True) * (1.0 / _NE)


def _vq_pallas(z3, wt, w2x, w2):
    return pl.pallas_call(
        _vq_body,
        grid=(_NSTEP,),
        in_specs=[
            pl.BlockSpec((_BLK, _D, _P), lambda b: (b, 0, 0)),
            pl.BlockSpec((_D, _NE), lambda b: (0, 0)),
            pl.BlockSpec((_NE, _D), lambda b: (0, 0)),
            pl.BlockSpec((_NE, 1), lambda b: (0, 0)),
        ],
        out_specs=[
            pl.BlockSpec((_BLK, _D, _P), lambda b: (b, 0, 0)),
            pl.BlockSpec((_BLK, 1, _P), lambda b: (b, 0, 0)),
            pl.BlockSpec((_NE, 1), lambda b: (0, 0)),
            pl.BlockSpec((1, 1), lambda b: (0, 0)),
            pl.BlockSpec((1, 1), lambda b: (0, 0)),
            pl.BlockSpec((1, 1), lambda b: (0, 0)),
        ],
        out_shape=[
            jax.ShapeDtypeStruct((_B, _D, _P), jnp.float32),
            jax.ShapeDtypeStruct((_B, 1, _P), jnp.int32),
            jax.ShapeDtypeStruct((_NE, 1), jnp.float32),
            jax.ShapeDtypeStruct((1, 1), jnp.float32),
            jax.ShapeDtypeStruct((1, 1), jnp.float32),
            jax.ShapeDtypeStruct((1, 1), jnp.float32),
        ],
    )(z3, wt, w2x, w2)


def kernel(z, W):
    z3 = z.reshape(_B, _D, _P)
    w2 = jnp.sum(W ** 2, axis=1, keepdims=True)
    qst3, idx3, _cnt, loss, perp, util = _vq_pallas(z3, W.T, W + W, w2)
    quantized_st = qst3.reshape(z.shape)
    encoding_indices = idx3.reshape(_NPIX)
    return (quantized_st, loss[0, 0], perp[0, 0], util[0, 0],
            encoding_indices)
